# baseline (speedup 1.0000x reference)
import functools
import math

import jax
import jax.numpy as jnp
from jax.experimental import pallas as pl
from jax.experimental.pallas import tpu as pltpu
from jax.experimental.pallas import tpu_sc as plsc

B, S, D = 8, 512, 64
K_CODE = 8192
HEADS = 8
DH = D // HEADS
KERNELS = (3, 5, 7, 9)
MAXPAD = max(KERNELS) // 2
TOK = B * S
CB_CHUNK = 2048


def _encoder_body(x_ref, pos_ref, wt_ref, cb4_ref, pjw_ref, pjb_ref,
                  aiw_ref, aib_ref, aow_ref, aob_ref, out_ref):
    x = x_ref[0] + pos_ref[...]
    zeros = jnp.zeros((MAXPAD, D), jnp.float32)
    xpad = jnp.concatenate([zeros, x, zeros], axis=0)

    ys = []
    off = 0
    for i, k in enumerate(KERNELS):
        p = k // 2
        acc = None
        for t in range(k):
            shift = xpad[MAXPAD - p + t:MAXPAD - p + t + S, :]
            part = jax.lax.dot(shift, wt_ref[off + t])
            acc = part if acc is None else acc + part
        off += k
        y = acc + cb4_ref[i][None, :]
        ys.append(jnp.maximum(y, 0.0))
    multi = jnp.concatenate(ys, axis=1)

    projected = jax.lax.dot(multi, pjw_ref[...]) + pjb_ref[...]
    qkv = jax.lax.dot(projected, aiw_ref[...]) + aib_ref[...]
    q = qkv[:, :D]
    k_ = qkv[:, D:2 * D]
    v = qkv[:, 2 * D:]

    scale = jnp.sqrt(jnp.float32(DH))
    os_ = []
    for h in range(HEADS):
        sl = slice(h * DH, (h + 1) * DH)
        qh, kh, vh = q[:, sl], k_[:, sl], v[:, sl]
        logits = jax.lax.dot_general(
            qh, kh, (((1,), (1,)), ((), ()))) / scale
        mx = jnp.max(logits, axis=-1, keepdims=True)
        e = jnp.exp(logits - mx)
        s = jnp.sum(e, axis=-1, keepdims=True)
        os_.append(jax.lax.dot(e, vh) / s)
    o = jnp.concatenate(os_, axis=1)

    attended = jax.lax.dot(o, aow_ref[...]) + aob_ref[...]
    out_ref[0] = projected + attended


def _run_encoder(x, params):
    wt = jnp.concatenate(
        [jnp.transpose(w, (2, 1, 0)) for w in params['conv_w']], axis=0)
    cb4 = jnp.stack(params['conv_b'], axis=0)
    pjw = params['proj_w'].T
    aiw = params['attn_in_w'].T
    aow = params['attn_out_w'].T

    specs = [
        pl.BlockSpec((1, S, D), lambda b: (b, 0, 0)),
        pl.BlockSpec((S, D), lambda b: (0, 0)),
        pl.BlockSpec((sum(KERNELS), D, D), lambda b: (0, 0, 0)),
        pl.BlockSpec((len(KERNELS), D), lambda b: (0, 0)),
        pl.BlockSpec((4 * D, D), lambda b: (0, 0)),
        pl.BlockSpec((1, D), lambda b: (0, 0)),
        pl.BlockSpec((D, 3 * D), lambda b: (0, 0)),
        pl.BlockSpec((1, 3 * D), lambda b: (0, 0)),
        pl.BlockSpec((D, D), lambda b: (0, 0)),
        pl.BlockSpec((1, D), lambda b: (0, 0)),
    ]
    return pl.pallas_call(
        _encoder_body,
        grid=(B,),
        in_specs=specs,
        out_specs=pl.BlockSpec((1, S, D), lambda b: (b, 0, 0)),
        out_shape=jax.ShapeDtypeStruct((B, S, D), jnp.float32),
        compiler_params=pltpu.CompilerParams(
            dimension_semantics=("parallel",)),
    )(x, params['pos'], wt, cb4, pjw, params['proj_b'][None, :],
      aiw, params['attn_in_b'][None, :], aow, params['attn_out_b'][None, :])


def _vq_body(flat_ref, x2_ref, cb_ref, idx_ref):
    flat = flat_ref[0]
    x2 = x2_ref[0]
    best_d = jnp.full((S,), jnp.inf, jnp.float32)
    best_i = jnp.zeros((S,), jnp.int32)
    for c0 in range(0, K_CODE, CB_CHUNK):
        cb = cb_ref[c0:c0 + CB_CHUNK, :]
        c2 = jnp.sum(cb ** 2, axis=1)
        m = jax.lax.dot_general(flat, cb, (((1,), (1,)), ((), ())))
        d = (x2 + c2[None, :]) - 2.0 * m
        loc_d = jnp.min(d, axis=1)
        iota = jax.lax.broadcasted_iota(jnp.int32, d.shape, 1)
        cand = jnp.where(d == loc_d[:, None], iota, K_CODE)
        loc_i = jnp.min(cand, axis=1) + c0
        upd = loc_d < best_d
        best_d = jnp.where(upd, loc_d, best_d)
        best_i = jnp.where(upd, loc_i, best_i)
    idx_ref[0, 0] = best_i


def _run_vq(flat, x2, codebook):
    n = flat.shape[0]
    return pl.pallas_call(
        _vq_body,
        grid=(n // S,),
        in_specs=[
            pl.BlockSpec((1, S, D), lambda t: (t, 0, 0)),
            pl.BlockSpec((1, S, 1), lambda t: (t, 0, 0)),
            pl.BlockSpec((K_CODE, D), lambda t: (0, 0)),
        ],
        out_specs=pl.BlockSpec((1, 1, S), lambda t: (t, 0, 0)),
        out_shape=jax.ShapeDtypeStruct((n // S, 1, S), jnp.int32),
        compiler_params=pltpu.CompilerParams(
            dimension_semantics=("parallel",)),
    )(flat.reshape(n // S, S, D), x2.reshape(n // S, S, 1), codebook)


_GATHER_W = 128


def _sc_gather(cb_pairs, idx_pair):
    n = idx_pair.shape[0]
    indices = idx_pair.reshape(1, n)
    mesh = plsc.VectorSubcoreMesh(
        core_axis_name="core", subcore_axis_name="subcore")

    @functools.partial(
        pl.kernel,
        out_type=jax.ShapeDtypeStruct((n, 128), cb_pairs.dtype),
        mesh=mesh)
    def kern(x_hbm, i_hbm, o_hbm):
        def body(i_vmem, o_vmem):
            pltpu.sync_copy(x_hbm.at[i_vmem.at[0]], o_vmem)

        pltpu.emit_pipeline(
            body,
            grid=(n // _GATHER_W,),
            in_specs=[pl.BlockSpec((1, _GATHER_W), index_map=lambda i: (0, i))],
            out_specs=[pl.BlockSpec((_GATHER_W, 128), index_map=lambda i: (i, 0))],
            core_axis_name='subcore',
            dimension_semantics=(pltpu.PARALLEL,),
        )(i_hbm, o_hbm)

    return kern(cb_pairs, indices)


def _finish_body(e_ref, ga_ref, gb_ref, idx_ref, hw_ref,
                 qst_ref, bits_ref, loss_ref):
    e = e_ref[...]
    g = jnp.concatenate([ga_ref[...], gb_ref[...]], axis=0)
    odd = (idx_ref[...] & 1) == 1
    q = jnp.where(odd, g[:, D:], g[:, :D])
    diff = q - e
    qst = e + diff
    qst_ref[...] = qst
    loss_ref[...] = (1.25 * jnp.mean(diff * diff)).reshape(1, 1)
    h = jax.lax.dot_general(qst, hw_ref[...], (((1,), (1,)), ((), ())))
    bits_ref[...] = (h > 0.0).astype(jnp.float32)


def _run_finish(flat, g_a, g_b, idx_col, hash_ws):
    hw = jnp.concatenate(hash_ws, axis=0)
    half = TOK // 2
    return pl.pallas_call(
        _finish_body,
        in_specs=[
            pl.BlockSpec((TOK, D), lambda: (0, 0)),
            pl.BlockSpec((half, 128), lambda: (0, 0)),
            pl.BlockSpec((half, 128), lambda: (0, 0)),
            pl.BlockSpec((TOK, 1), lambda: (0, 0)),
            pl.BlockSpec((D, D), lambda: (0, 0)),
        ],
        out_specs=[
            pl.BlockSpec((TOK, D), lambda: (0, 0)),
            pl.BlockSpec((TOK, D), lambda: (0, 0)),
            pl.BlockSpec((1, 1), lambda: (0, 0)),
        ],
        out_shape=[
            jax.ShapeDtypeStruct((TOK, D), jnp.float32),
            jax.ShapeDtypeStruct((TOK, D), jnp.float32),
            jax.ShapeDtypeStruct((1, 1), jnp.float32),
        ],
    )(flat, g_a, g_b, idx_col, hw)


def kernel(inputs, params):
    encoded = _run_encoder(inputs, params)
    flat = encoded.reshape(TOK, D)
    x2 = jnp.sum(flat ** 2, axis=1, keepdims=True)
    cb_pairs = params['codebook'].reshape(K_CODE // 2, 2 * D)
    half = TOK // 2
    idx_a = _run_vq(flat[:half], x2[:half], params['codebook'])
    g_a = _sc_gather(cb_pairs, jnp.right_shift(idx_a.reshape(half), 1))
    idx_b = _run_vq(flat[half:], x2[half:], params['codebook'])
    g_b = _sc_gather(cb_pairs, jnp.right_shift(idx_b.reshape(half), 1))
    idx = jnp.concatenate([idx_a, idx_b], axis=0)
    qst, bits, loss = _run_finish(
        flat, g_a, g_b, idx.reshape(TOK, 1), params['hash_w'])
    hash_codes = bits.reshape(B, S, len(params['hash_w']), -1)
    return (qst.reshape(B, S, D), loss.reshape(()), idx.reshape(B, S),
            hash_codes)


# speedup vs baseline: 1.0251x; 1.0251x over previous
import functools
import math

import jax
import jax.numpy as jnp
from jax.experimental import pallas as pl
from jax.experimental.pallas import tpu as pltpu
from jax.experimental.pallas import tpu_sc as plsc

B, S, D = 8, 512, 64
K_CODE = 8192
HEADS = 8
DH = D // HEADS
KERNELS = (3, 5, 7, 9)
MAXPAD = max(KERNELS) // 2
TOK = B * S
CB_CHUNK = 2048


def _encoder_body(x_ref, pos_ref, wt_ref, cb4_ref, pjw_ref, pjb_ref,
                  aiw_ref, aib_ref, aow_ref, aob_ref, out_ref):
    x = x_ref[0] + pos_ref[...]
    zeros = jnp.zeros((MAXPAD, D), jnp.float32)
    xpad = jnp.concatenate([zeros, x, zeros], axis=0)

    ys = []
    off = 0
    for i, k in enumerate(KERNELS):
        p = k // 2
        acc = None
        for t in range(k):
            shift = xpad[MAXPAD - p + t:MAXPAD - p + t + S, :]
            part = jax.lax.dot(shift, wt_ref[off + t])
            acc = part if acc is None else acc + part
        off += k
        y = acc + cb4_ref[i][None, :]
        ys.append(jnp.maximum(y, 0.0))
    multi = jnp.concatenate(ys, axis=1)

    projected = jax.lax.dot(multi, pjw_ref[...]) + pjb_ref[...]
    qkv = jax.lax.dot(projected, aiw_ref[...]) + aib_ref[...]
    q = qkv[:, :D]
    k_ = qkv[:, D:2 * D]
    v = qkv[:, 2 * D:]

    scale = jnp.sqrt(jnp.float32(DH))
    os_ = []
    for h in range(HEADS):
        sl = slice(h * DH, (h + 1) * DH)
        qh, kh, vh = q[:, sl], k_[:, sl], v[:, sl]
        logits = jax.lax.dot_general(
            qh, kh, (((1,), (1,)), ((), ()))) / scale
        mx = jnp.max(logits, axis=-1, keepdims=True)
        e = jnp.exp(logits - mx)
        s = jnp.sum(e, axis=-1, keepdims=True)
        os_.append(jax.lax.dot(e, vh) / s)
    o = jnp.concatenate(os_, axis=1)

    attended = jax.lax.dot(o, aow_ref[...]) + aob_ref[...]
    out_ref[0] = projected + attended


def _run_encoder(x, params):
    wt = jnp.concatenate(
        [jnp.transpose(w, (2, 1, 0)) for w in params['conv_w']], axis=0)
    cb4 = jnp.stack(params['conv_b'], axis=0)
    pjw = params['proj_w'].T
    aiw = params['attn_in_w'].T
    aow = params['attn_out_w'].T

    specs = [
        pl.BlockSpec((1, S, D), lambda b: (b, 0, 0)),
        pl.BlockSpec((S, D), lambda b: (0, 0)),
        pl.BlockSpec((sum(KERNELS), D, D), lambda b: (0, 0, 0)),
        pl.BlockSpec((len(KERNELS), D), lambda b: (0, 0)),
        pl.BlockSpec((4 * D, D), lambda b: (0, 0)),
        pl.BlockSpec((1, D), lambda b: (0, 0)),
        pl.BlockSpec((D, 3 * D), lambda b: (0, 0)),
        pl.BlockSpec((1, 3 * D), lambda b: (0, 0)),
        pl.BlockSpec((D, D), lambda b: (0, 0)),
        pl.BlockSpec((1, D), lambda b: (0, 0)),
    ]
    return pl.pallas_call(
        _encoder_body,
        grid=(B,),
        in_specs=specs,
        out_specs=pl.BlockSpec((1, S, D), lambda b: (b, 0, 0)),
        out_shape=jax.ShapeDtypeStruct((B, S, D), jnp.float32),
        compiler_params=pltpu.CompilerParams(
            dimension_semantics=("parallel",)),
    )(x, params['pos'], wt, cb4, pjw, params['proj_b'][None, :],
      aiw, params['attn_in_b'][None, :], aow, params['attn_out_b'][None, :])


def _vq_body(flat_ref, x2_ref, cb_ref, idx_ref):
    flat = flat_ref[0]
    x2 = x2_ref[0]
    best_d = jnp.full((S,), jnp.inf, jnp.float32)
    best_i = jnp.zeros((S,), jnp.int32)
    for c0 in range(0, K_CODE, CB_CHUNK):
        cb = cb_ref[c0:c0 + CB_CHUNK, :]
        c2 = jnp.sum(cb ** 2, axis=1)
        m = jax.lax.dot_general(flat, cb, (((1,), (1,)), ((), ())))
        d = (x2 + c2[None, :]) - 2.0 * m
        loc_d = jnp.min(d, axis=1)
        iota = jax.lax.broadcasted_iota(jnp.int32, d.shape, 1)
        cand = jnp.where(d == loc_d[:, None], iota, K_CODE)
        loc_i = jnp.min(cand, axis=1) + c0
        upd = loc_d < best_d
        best_d = jnp.where(upd, loc_d, best_d)
        best_i = jnp.where(upd, loc_i, best_i)
    idx_ref[0, 0] = best_i


def _run_vq(flat, x2, codebook):
    n = flat.shape[0]
    return pl.pallas_call(
        _vq_body,
        grid=(n // S,),
        in_specs=[
            pl.BlockSpec((1, S, D), lambda t: (t, 0, 0)),
            pl.BlockSpec((1, S, 1), lambda t: (t, 0, 0)),
            pl.BlockSpec((K_CODE, D), lambda t: (0, 0)),
        ],
        out_specs=pl.BlockSpec((1, 1, S), lambda t: (t, 0, 0)),
        out_shape=jax.ShapeDtypeStruct((n // S, 1, S), jnp.int32),
        compiler_params=pltpu.CompilerParams(
            dimension_semantics=("parallel",)),
    )(flat.reshape(n // S, S, D), x2.reshape(n // S, S, 1), codebook)


_GATHER_W = 128


def _sc_gather(cb_pairs, idx_pair):
    n = idx_pair.shape[0]
    indices = idx_pair.reshape(1, n)
    mesh = plsc.VectorSubcoreMesh(
        core_axis_name="core", subcore_axis_name="subcore")

    @functools.partial(
        pl.kernel,
        out_type=jax.ShapeDtypeStruct((n, 128), cb_pairs.dtype),
        mesh=mesh)
    def kern(x_hbm, i_hbm, o_hbm):
        def body(i_vmem, o_vmem):
            pltpu.sync_copy(x_hbm.at[i_vmem.at[0]], o_vmem)

        pltpu.emit_pipeline(
            body,
            grid=(n // _GATHER_W,),
            in_specs=[pl.BlockSpec((1, _GATHER_W), index_map=lambda i: (0, i))],
            out_specs=[pl.BlockSpec((_GATHER_W, 128), index_map=lambda i: (i, 0))],
            core_axis_name='subcore',
            dimension_semantics=(pltpu.PARALLEL,),
        )(i_hbm, o_hbm)

    return kern(cb_pairs, indices)


def _finish_body(e_ref, g_ref, idx_ref, hw_ref,
                 qst_ref, bits_ref, loss_ref):
    e = e_ref[...]
    g = g_ref[...]
    odd = (idx_ref[...] & 1) == 1
    q = jnp.where(odd, g[:, D:], g[:, :D])
    diff = q - e
    qst = e + diff
    qst_ref[...] = qst
    loss_ref[...] = (1.25 * jnp.mean(diff * diff)).reshape(1, 1)
    h = jax.lax.dot_general(qst, hw_ref[...], (((1,), (1,)), ((), ())))
    bits_ref[...] = (h > 0.0).astype(jnp.float32)


def _run_finish(flat, g, idx_col, hash_ws):
    hw = jnp.concatenate(hash_ws, axis=0)
    return pl.pallas_call(
        _finish_body,
        in_specs=[
            pl.BlockSpec((TOK, D), lambda: (0, 0)),
            pl.BlockSpec((TOK, 128), lambda: (0, 0)),
            pl.BlockSpec((TOK, 1), lambda: (0, 0)),
            pl.BlockSpec((D, D), lambda: (0, 0)),
        ],
        out_specs=[
            pl.BlockSpec((TOK, D), lambda: (0, 0)),
            pl.BlockSpec((TOK, D), lambda: (0, 0)),
            pl.BlockSpec((1, 1), lambda: (0, 0)),
        ],
        out_shape=[
            jax.ShapeDtypeStruct((TOK, D), jnp.float32),
            jax.ShapeDtypeStruct((TOK, D), jnp.float32),
            jax.ShapeDtypeStruct((1, 1), jnp.float32),
        ],
    )(flat, g, idx_col, hw)


def kernel(inputs, params):
    encoded = _run_encoder(inputs, params)
    flat = encoded.reshape(TOK, D)
    x2 = jnp.sum(flat ** 2, axis=1, keepdims=True)
    cb_pairs = params['codebook'].reshape(K_CODE // 2, 2 * D)
    idx = _run_vq(flat, x2, params['codebook'])
    g = _sc_gather(cb_pairs, jnp.right_shift(idx.reshape(TOK), 1))
    qst, bits, loss = _run_finish(
        flat, g, idx.reshape(TOK, 1), params['hash_w'])
    hash_codes = bits.reshape(B, S, len(params['hash_w']), -1)
    return (qst.reshape(B, S, D), loss.reshape(()), idx.reshape(B, S),
            hash_codes)


# speedup vs baseline: 1.0821x; 1.0556x over previous
import functools
import math

import jax
import jax.numpy as jnp
from jax.experimental import pallas as pl
from jax.experimental.pallas import tpu as pltpu
from jax.experimental.pallas import tpu_sc as plsc

B, S, D = 8, 512, 64
K_CODE = 8192
HEADS = 8
DH = D // HEADS
KERNELS = (3, 5, 7, 9)
MAXPAD = max(KERNELS) // 2
TOK = B * S
CB_CHUNK = 2048


def _encoder_body(x_ref, pos_ref, wt_ref, cb4_ref, pjw_ref, pjb_ref,
                  aiw_ref, aib_ref, aow_ref, aob_ref, out_ref):
    x = x_ref[0] + pos_ref[...]
    zeros = jnp.zeros((MAXPAD, D), jnp.float32)
    xpad = jnp.concatenate([zeros, x, zeros], axis=0)

    ys = []
    off = 0
    for i, k in enumerate(KERNELS):
        p = k // 2
        acc = None
        for t in range(k):
            shift = xpad[MAXPAD - p + t:MAXPAD - p + t + S, :]
            part = jax.lax.dot(shift, wt_ref[off + t])
            acc = part if acc is None else acc + part
        off += k
        y = acc + cb4_ref[i][None, :]
        ys.append(jnp.maximum(y, 0.0))
    multi = jnp.concatenate(ys, axis=1)

    projected = jax.lax.dot(multi, pjw_ref[...]) + pjb_ref[...]
    qkv = jax.lax.dot(projected, aiw_ref[...]) + aib_ref[...]
    q = qkv[:, :D]
    k_ = qkv[:, D:2 * D]
    v = qkv[:, 2 * D:]

    scale = jnp.sqrt(jnp.float32(DH))
    os_ = []
    for h in range(HEADS):
        sl = slice(h * DH, (h + 1) * DH)
        qh, kh, vh = q[:, sl], k_[:, sl], v[:, sl]
        logits = jax.lax.dot_general(
            qh, kh, (((1,), (1,)), ((), ()))) / scale
        mx = jnp.max(logits, axis=-1, keepdims=True)
        e = jnp.exp(logits - mx)
        s = jnp.sum(e, axis=-1, keepdims=True)
        os_.append(jax.lax.dot(e, vh) / s)
    o = jnp.concatenate(os_, axis=1)

    attended = jax.lax.dot(o, aow_ref[...]) + aob_ref[...]
    out_ref[0] = projected + attended


def _run_encoder(x, params):
    wt = jnp.concatenate(
        [jnp.transpose(w, (2, 1, 0)) for w in params['conv_w']], axis=0)
    cb4 = jnp.stack(params['conv_b'], axis=0)
    pjw = params['proj_w'].T
    aiw = params['attn_in_w'].T
    aow = params['attn_out_w'].T

    specs = [
        pl.BlockSpec((1, S, D), lambda b: (b, 0, 0)),
        pl.BlockSpec((S, D), lambda b: (0, 0)),
        pl.BlockSpec((sum(KERNELS), D, D), lambda b: (0, 0, 0)),
        pl.BlockSpec((len(KERNELS), D), lambda b: (0, 0)),
        pl.BlockSpec((4 * D, D), lambda b: (0, 0)),
        pl.BlockSpec((1, D), lambda b: (0, 0)),
        pl.BlockSpec((D, 3 * D), lambda b: (0, 0)),
        pl.BlockSpec((1, 3 * D), lambda b: (0, 0)),
        pl.BlockSpec((D, D), lambda b: (0, 0)),
        pl.BlockSpec((1, D), lambda b: (0, 0)),
    ]
    return pl.pallas_call(
        _encoder_body,
        grid=(B,),
        in_specs=specs,
        out_specs=pl.BlockSpec((1, S, D), lambda b: (b, 0, 0)),
        out_shape=jax.ShapeDtypeStruct((B, S, D), jnp.float32),
        compiler_params=pltpu.CompilerParams(
            dimension_semantics=("parallel",)),
    )(x, params['pos'], wt, cb4, pjw, params['proj_b'][None, :],
      aiw, params['attn_in_b'][None, :], aow, params['attn_out_b'][None, :])


def _vq_body(flat_ref, x2_ref, cb_ref, idx_ref):
    flat = flat_ref[0]
    x2 = x2_ref[0]
    best_d = jnp.full((S,), jnp.inf, jnp.float32)
    best_i = jnp.zeros((S,), jnp.int32)
    for c0 in range(0, K_CODE, CB_CHUNK):
        cb = cb_ref[c0:c0 + CB_CHUNK, :]
        c2 = jnp.sum(cb ** 2, axis=1)
        m = jax.lax.dot_general(flat, cb, (((1,), (1,)), ((), ())))
        d = (x2 + c2[None, :]) - 2.0 * m
        loc_d = jnp.min(d, axis=1)
        iota = jax.lax.broadcasted_iota(jnp.int32, d.shape, 1)
        cand = jnp.where(d == loc_d[:, None], iota, K_CODE)
        loc_i = jnp.min(cand, axis=1) + c0
        upd = loc_d < best_d
        best_d = jnp.where(upd, loc_d, best_d)
        best_i = jnp.where(upd, loc_i, best_i)
    idx_ref[0, 0] = best_i


def _run_vq(flat, x2, codebook):
    n = flat.shape[0]
    return pl.pallas_call(
        _vq_body,
        grid=(n // S,),
        in_specs=[
            pl.BlockSpec((1, S, D), lambda t: (t, 0, 0)),
            pl.BlockSpec((1, S, 1), lambda t: (t, 0, 0)),
            pl.BlockSpec((K_CODE, D), lambda t: (0, 0)),
        ],
        out_specs=pl.BlockSpec((1, 1, S), lambda t: (t, 0, 0)),
        out_shape=jax.ShapeDtypeStruct((n // S, 1, S), jnp.int32),
        compiler_params=pltpu.CompilerParams(
            dimension_semantics=("parallel",)),
    )(flat.reshape(n // S, S, D), x2.reshape(n // S, S, 1), codebook)


_GATHER_W = 128


def _sc_gather(cb_wide, idx):
    n = idx.shape[0]
    indices = idx.reshape(1, n)
    mesh = plsc.VectorSubcoreMesh(
        core_axis_name="core", subcore_axis_name="subcore")

    @functools.partial(
        pl.kernel,
        out_type=jax.ShapeDtypeStruct((n, 128), cb_wide.dtype),
        mesh=mesh)
    def kern(x_hbm, i_hbm, o_hbm):
        def body(i_vmem, o_vmem):
            pltpu.sync_copy(x_hbm.at[i_vmem.at[0]], o_vmem)

        pltpu.emit_pipeline(
            body,
            grid=(n // _GATHER_W,),
            in_specs=[pl.BlockSpec((1, _GATHER_W), index_map=lambda i: (0, i))],
            out_specs=[pl.BlockSpec((_GATHER_W, 128), index_map=lambda i: (i, 0))],
            core_axis_name='subcore',
            dimension_semantics=(pltpu.PARALLEL,),
        )(i_hbm, o_hbm)

    return kern(cb_wide, indices)


def _finish_body(e_ref, g_ref, hw_ref, qst_ref, bits_ref, loss_ref):
    e = e_ref[...]
    q = g_ref[:, :D]
    diff = q - e
    qst = e + diff
    qst_ref[...] = qst
    loss_ref[...] = (1.25 * jnp.mean(diff * diff)).reshape(1, 1)
    h = jax.lax.dot_general(qst, hw_ref[...], (((1,), (1,)), ((), ())))
    bits_ref[...] = (h > 0.0).astype(jnp.float32)


def _run_finish(flat, g, hash_ws):
    hw = jnp.concatenate(hash_ws, axis=0)
    return pl.pallas_call(
        _finish_body,
        in_specs=[
            pl.BlockSpec((TOK, D), lambda: (0, 0)),
            pl.BlockSpec((TOK, 128), lambda: (0, 0)),
            pl.BlockSpec((D, D), lambda: (0, 0)),
        ],
        out_specs=[
            pl.BlockSpec((TOK, D), lambda: (0, 0)),
            pl.BlockSpec((TOK, D), lambda: (0, 0)),
            pl.BlockSpec((1, 1), lambda: (0, 0)),
        ],
        out_shape=[
            jax.ShapeDtypeStruct((TOK, D), jnp.float32),
            jax.ShapeDtypeStruct((TOK, D), jnp.float32),
            jax.ShapeDtypeStruct((1, 1), jnp.float32),
        ],
    )(flat, g, hw)


def kernel(inputs, params):
    encoded = _run_encoder(inputs, params)
    flat = encoded.reshape(TOK, D)
    x2 = jnp.sum(flat ** 2, axis=1, keepdims=True)
    cb_wide = jnp.pad(params['codebook'], ((0, 0), (0, 128 - D)))
    idx = _run_vq(flat, x2, params['codebook'])
    g = _sc_gather(cb_wide, idx.reshape(TOK))
    qst, bits, loss = _run_finish(flat, g, params['hash_w'])
    hash_codes = bits.reshape(B, S, len(params['hash_w']), -1)
    return (qst.reshape(B, S, D), loss.reshape(()), idx.reshape(B, S),
            hash_codes)


# speedup vs baseline: 1.1030x; 1.0193x over previous
import functools
import math

import jax
import jax.numpy as jnp
from jax.experimental import pallas as pl
from jax.experimental.pallas import tpu as pltpu
from jax.experimental.pallas import tpu_sc as plsc

B, S, D = 8, 512, 64
K_CODE = 8192
HEADS = 8
DH = D // HEADS
KERNELS = (3, 5, 7, 9)
MAXPAD = max(KERNELS) // 2
TOK = B * S
CB_CHUNK = 2048


def _encoder_body(x_ref, pos_ref, wt_ref, cb4_ref, pjw_ref, pjb_ref,
                  aiw_ref, aib_ref, aow_ref, aob_ref, out_ref):
    x = x_ref[0] + pos_ref[...]
    zeros = jnp.zeros((MAXPAD, D), jnp.float32)
    xpad = jnp.concatenate([zeros, x, zeros], axis=0)

    ys = []
    off = 0
    for i, k in enumerate(KERNELS):
        p = k // 2
        xcat = jnp.concatenate(
            [xpad[MAXPAD - p + t:MAXPAD - p + t + S, :] for t in range(k)],
            axis=1)
        w = wt_ref[off:off + k].reshape(k * D, D)
        off += k
        y = jax.lax.dot(xcat, w) + cb4_ref[i][None, :]
        ys.append(jnp.maximum(y, 0.0))
    multi = jnp.concatenate(ys, axis=1)

    projected = jax.lax.dot(multi, pjw_ref[...]) + pjb_ref[...]
    qkv = jax.lax.dot(projected, aiw_ref[...]) + aib_ref[...]
    q = qkv[:, :D]
    k_ = qkv[:, D:2 * D]
    v = qkv[:, 2 * D:]

    scale = jnp.sqrt(jnp.float32(DH))
    os_ = []
    for h in range(HEADS):
        sl = slice(h * DH, (h + 1) * DH)
        qh, kh, vh = q[:, sl], k_[:, sl], v[:, sl]
        logits = jax.lax.dot_general(
            qh, kh, (((1,), (1,)), ((), ()))) / scale
        mx = jnp.max(logits, axis=-1, keepdims=True)
        e = jnp.exp(logits - mx)
        s = jnp.sum(e, axis=-1, keepdims=True)
        os_.append(jax.lax.dot(e, vh) / s)
    o = jnp.concatenate(os_, axis=1)

    attended = jax.lax.dot(o, aow_ref[...]) + aob_ref[...]
    out_ref[0] = projected + attended


def _run_encoder(x, params):
    wt = jnp.concatenate(
        [jnp.transpose(w, (2, 1, 0)) for w in params['conv_w']], axis=0)
    cb4 = jnp.stack(params['conv_b'], axis=0)
    pjw = params['proj_w'].T
    aiw = params['attn_in_w'].T
    aow = params['attn_out_w'].T

    specs = [
        pl.BlockSpec((1, S, D), lambda b: (b, 0, 0)),
        pl.BlockSpec((S, D), lambda b: (0, 0)),
        pl.BlockSpec((sum(KERNELS), D, D), lambda b: (0, 0, 0)),
        pl.BlockSpec((len(KERNELS), D), lambda b: (0, 0)),
        pl.BlockSpec((4 * D, D), lambda b: (0, 0)),
        pl.BlockSpec((1, D), lambda b: (0, 0)),
        pl.BlockSpec((D, 3 * D), lambda b: (0, 0)),
        pl.BlockSpec((1, 3 * D), lambda b: (0, 0)),
        pl.BlockSpec((D, D), lambda b: (0, 0)),
        pl.BlockSpec((1, D), lambda b: (0, 0)),
    ]
    return pl.pallas_call(
        _encoder_body,
        grid=(B,),
        in_specs=specs,
        out_specs=pl.BlockSpec((1, S, D), lambda b: (b, 0, 0)),
        out_shape=jax.ShapeDtypeStruct((B, S, D), jnp.float32),
        compiler_params=pltpu.CompilerParams(
            dimension_semantics=("parallel",)),
    )(x, params['pos'], wt, cb4, pjw, params['proj_b'][None, :],
      aiw, params['attn_in_b'][None, :], aow, params['attn_out_b'][None, :])


def _vq_body(flat_ref, x2_ref, cb_ref, idx_ref):
    flat = flat_ref[0]
    x2 = x2_ref[0]
    best_d = jnp.full((S,), jnp.inf, jnp.float32)
    best_i = jnp.zeros((S,), jnp.int32)
    for c0 in range(0, K_CODE, CB_CHUNK):
        cb = cb_ref[c0:c0 + CB_CHUNK, :]
        c2 = jnp.sum(cb ** 2, axis=1)
        m = jax.lax.dot_general(flat, cb, (((1,), (1,)), ((), ())))
        d = (x2 + c2[None, :]) - 2.0 * m
        loc_d = jnp.min(d, axis=1)
        iota = jax.lax.broadcasted_iota(jnp.int32, d.shape, 1)
        cand = jnp.where(d == loc_d[:, None], iota, K_CODE)
        loc_i = jnp.min(cand, axis=1) + c0
        upd = loc_d < best_d
        best_d = jnp.where(upd, loc_d, best_d)
        best_i = jnp.where(upd, loc_i, best_i)
    idx_ref[0, 0] = best_i


def _run_vq(flat, x2, codebook):
    n = flat.shape[0]
    return pl.pallas_call(
        _vq_body,
        grid=(n // S,),
        in_specs=[
            pl.BlockSpec((1, S, D), lambda t: (t, 0, 0)),
            pl.BlockSpec((1, S, 1), lambda t: (t, 0, 0)),
            pl.BlockSpec((K_CODE, D), lambda t: (0, 0)),
        ],
        out_specs=pl.BlockSpec((1, 1, S), lambda t: (t, 0, 0)),
        out_shape=jax.ShapeDtypeStruct((n // S, 1, S), jnp.int32),
        compiler_params=pltpu.CompilerParams(
            dimension_semantics=("parallel",)),
    )(flat.reshape(n // S, S, D), x2.reshape(n // S, S, 1), codebook)


_GATHER_W = 128


def _sc_gather(cb_wide, idx):
    n = idx.shape[0]
    indices = idx.reshape(1, n)
    mesh = plsc.VectorSubcoreMesh(
        core_axis_name="core", subcore_axis_name="subcore")

    @functools.partial(
        pl.kernel,
        out_type=jax.ShapeDtypeStruct((n, 128), cb_wide.dtype),
        mesh=mesh)
    def kern(x_hbm, i_hbm, o_hbm):
        def body(i_vmem, o_vmem):
            pltpu.sync_copy(x_hbm.at[i_vmem.at[0]], o_vmem)

        pltpu.emit_pipeline(
            body,
            grid=(n // _GATHER_W,),
            in_specs=[pl.BlockSpec((1, _GATHER_W), index_map=lambda i: (0, i))],
            out_specs=[pl.BlockSpec((_GATHER_W, 128), index_map=lambda i: (i, 0))],
            core_axis_name='subcore',
            dimension_semantics=(pltpu.PARALLEL,),
        )(i_hbm, o_hbm)

    return kern(cb_wide, indices)


def _finish_body(e_ref, g_ref, hw_ref, qst_ref, bits_ref, loss_ref):
    e = e_ref[...]
    q = g_ref[:, :D]
    diff = q - e
    qst = e + diff
    qst_ref[...] = qst
    loss_ref[...] = (1.25 * jnp.mean(diff * diff)).reshape(1, 1)
    h = jax.lax.dot_general(qst, hw_ref[...], (((1,), (1,)), ((), ())))
    bits_ref[...] = (h > 0.0).astype(jnp.float32)


def _run_finish(flat, g, hash_ws):
    hw = jnp.concatenate(hash_ws, axis=0)
    return pl.pallas_call(
        _finish_body,
        in_specs=[
            pl.BlockSpec((TOK, D), lambda: (0, 0)),
            pl.BlockSpec((TOK, 128), lambda: (0, 0)),
            pl.BlockSpec((D, D), lambda: (0, 0)),
        ],
        out_specs=[
            pl.BlockSpec((TOK, D), lambda: (0, 0)),
            pl.BlockSpec((TOK, D), lambda: (0, 0)),
            pl.BlockSpec((1, 1), lambda: (0, 0)),
        ],
        out_shape=[
            jax.ShapeDtypeStruct((TOK, D), jnp.float32),
            jax.ShapeDtypeStruct((TOK, D), jnp.float32),
            jax.ShapeDtypeStruct((1, 1), jnp.float32),
        ],
    )(flat, g, hw)


def kernel(inputs, params):
    encoded = _run_encoder(inputs, params)
    flat = encoded.reshape(TOK, D)
    x2 = jnp.sum(flat ** 2, axis=1, keepdims=True)
    cb_wide = jnp.pad(params['codebook'], ((0, 0), (0, 128 - D)))
    idx = _run_vq(flat, x2, params['codebook'])
    g = _sc_gather(cb_wide, idx.reshape(TOK))
    qst, bits, loss = _run_finish(flat, g, params['hash_w'])
    hash_codes = bits.reshape(B, S, len(params['hash_w']), -1)
    return (qst.reshape(B, S, D), loss.reshape(()), idx.reshape(B, S),
            hash_codes)
